# baseline (speedup 1.0000x reference)
import numpy as np
import jax
import jax.numpy as jnp
from jax.experimental import pallas as pl

_L = 6
_C = 128
_NO = (_L + 1) ** 2


def _so2_pair_table():
    ret = []
    widx = 0
    for lo in range(_L + 1):
        for li in range(_L + 1):
            mmax = min(lo, li)
            for mw in range(-mmax, mmax + 1):
                if mw != 0:
                    prs = ((-abs(mw), -mw), (abs(mw), mw))
                else:
                    prs = ((0, 0),)
                for mo, mi in prs:
                    ret.append((lo * lo + mo + lo, li * li + mi + li,
                                -1.0 if (mo > 0 and mi < 0) else 1.0, widx))
                widx += 1
    ret.sort()
    return ret, widx


_PAIRS, _NW = _so2_pair_table()
_PAIR_LUT = {(mi, mo): (s, w) for mo, mi, s, w in _PAIRS}

_GRP = []
for _m in range(_L + 1):
    if _m == 0:
        _GRP.append([l * l + l for l in range(_L + 1)])
    else:
        g = []
        for l in range(_m, _L + 1):
            g.append(l * l + l - _m)
            g.append(l * l + l + _m)
        _GRP.append(g)
_GK = [len(g) * _C for g in _GRP]


def _wprep_body(w_ref, *o_refs):
    for m, g in enumerate(_GRP):
        for r, oi in enumerate(g):
            for c, oo in enumerate(g):
                s, wi = _PAIR_LUT[(oi, oo)]
                wb = w_ref[wi]
                if s < 0:
                    wb = -wb
                o_refs[m][r * _C:(r + 1) * _C, c * _C:(c + 1) * _C] = (
                    wb.astype(jnp.bfloat16))


def _so2_body(x_ref, *refs):
    w_refs = refs[:_L + 1]
    o_ref = refs[_L + 1]
    dn = (((1,), (0,)), ((), ()))
    for m, g in enumerate(_GRP):
        acc = None
        for r, oi in enumerate(g):
            xb = x_ref[:, oi * _C:(oi + 1) * _C].astype(jnp.bfloat16)
            d = jax.lax.dot_general(xb, w_refs[m][r * _C:(r + 1) * _C, :],
                                    dn, preferred_element_type=jnp.float32)
            acc = d if acc is None else acc + d
        for c, oo in enumerate(g):
            o_ref[:, oo * _C:(oo + 1) * _C] = acc[:, c * _C:(c + 1) * _C]


def kernel(x, weight):
    n = x.shape[0]
    tn = 256
    xf = x.reshape(n, _NO * _C)
    wf = weight.reshape(_NW, _C, _C)
    wgrp = pl.pallas_call(
        _wprep_body,
        in_specs=[pl.BlockSpec((_NW, _C, _C), lambda: (0, 0, 0))],
        out_specs=[pl.BlockSpec((k, k), lambda: (0, 0)) for k in _GK],
        out_shape=[jax.ShapeDtypeStruct((k, k), jnp.bfloat16) for k in _GK],
    )(wf)
    out = pl.pallas_call(
        _so2_body,
        grid=(n // tn,),
        in_specs=[pl.BlockSpec((tn, _NO * _C), lambda i: (i, 0))]
        + [pl.BlockSpec((k, k), lambda i: (0, 0)) for k in _GK],
        out_specs=pl.BlockSpec((tn, _NO * _C), lambda i: (i, 0)),
        out_shape=jax.ShapeDtypeStruct((n, _NO * _C), jnp.float32),
    )(xf, *wgrp)
    return out.reshape(n, _NO, _C)


# speedup vs baseline: 1.2341x; 1.2341x over previous
import numpy as np
import jax
import jax.numpy as jnp
from jax.experimental import pallas as pl

_L = 6
_C = 128
_NO = (_L + 1) ** 2


def _so2_pair_table():
    ret = []
    widx = 0
    for lo in range(_L + 1):
        for li in range(_L + 1):
            mmax = min(lo, li)
            for mw in range(-mmax, mmax + 1):
                if mw != 0:
                    prs = ((-abs(mw), -mw), (abs(mw), mw))
                else:
                    prs = ((0, 0),)
                for mo, mi in prs:
                    ret.append((lo * lo + mo + lo, li * li + mi + li,
                                -1.0 if (mo > 0 and mi < 0) else 1.0, widx))
                widx += 1
    ret.sort()
    return ret, widx


_PAIRS, _NW = _so2_pair_table()
_PAIR_LUT = {(mi, mo): (s, w) for mo, mi, s, w in _PAIRS}

_GRP = []
for _m in range(_L + 1):
    if _m == 0:
        _GRP.append([l * l + l for l in range(_L + 1)])
    else:
        g = []
        for l in range(_m, _L + 1):
            g.append(l * l + l - _m)
            g.append(l * l + l + _m)
        _GRP.append(g)
_GK = [len(g) * _C for g in _GRP]


def _wprep_body(w_ref, *o_refs):
    for m, g in enumerate(_GRP):
        for r, oi in enumerate(g):
            for c, oo in enumerate(g):
                s, wi = _PAIR_LUT[(oi, oo)]
                wb = w_ref[0, wi]
                if s < 0:
                    wb = -wb
                o_refs[m][r * _C:(r + 1) * _C, c * _C:(c + 1) * _C] = (
                    wb.astype(jnp.bfloat16))


def _so2_body(x_ref, *refs):
    w_refs = refs[:_L + 1]
    o_ref = refs[_L + 1]
    dn = (((1,), (0,)), ((), ()))
    for m, g in enumerate(_GRP):
        acc = None
        for r, oi in enumerate(g):
            xb = x_ref[:, oi, :].astype(jnp.bfloat16)
            d = jax.lax.dot_general(xb, w_refs[m][r * _C:(r + 1) * _C, :],
                                    dn, preferred_element_type=jnp.float32)
            acc = d if acc is None else acc + d
        for c, oo in enumerate(g):
            o_ref[:, oo, :] = acc[:, c * _C:(c + 1) * _C]


def kernel(x, weight):
    n = x.shape[0]
    tn = 256
    wgrp = pl.pallas_call(
        _wprep_body,
        in_specs=[pl.BlockSpec((1, _NW, _C, _C), lambda: (0, 0, 0, 0))],
        out_specs=[pl.BlockSpec((k, k), lambda: (0, 0)) for k in _GK],
        out_shape=[jax.ShapeDtypeStruct((k, k), jnp.bfloat16) for k in _GK],
    )(weight)
    out = pl.pallas_call(
        _so2_body,
        grid=(n // tn,),
        in_specs=[pl.BlockSpec((tn, _NO, _C), lambda i: (i, 0, 0))]
        + [pl.BlockSpec((k, k), lambda i: (0, 0)) for k in _GK],
        out_specs=pl.BlockSpec((tn, _NO, _C), lambda i: (i, 0, 0)),
        out_shape=jax.ShapeDtypeStruct((n, _NO, _C), jnp.float32),
    )(x, *wgrp)
    return out


# speedup vs baseline: 2.5601x; 2.0744x over previous
import numpy as np
import jax
import jax.numpy as jnp
from jax.experimental import pallas as pl
from jax.experimental.pallas import tpu as pltpu

_L = 6
_C = 128
_NO = (_L + 1) ** 2


def _so2_pair_table():
    ret = []
    widx = 0
    for lo in range(_L + 1):
        for li in range(_L + 1):
            mmax = min(lo, li)
            for mw in range(-mmax, mmax + 1):
                if mw != 0:
                    prs = ((-abs(mw), -mw), (abs(mw), mw))
                else:
                    prs = ((0, 0),)
                for mo, mi in prs:
                    ret.append((lo * lo + mo + lo, li * li + mi + li,
                                -1.0 if (mo > 0 and mi < 0) else 1.0, widx))
                widx += 1
    ret.sort()
    return ret, widx


_PAIRS, _NW = _so2_pair_table()
_PAIR_LUT = {(mi, mo): (s, w) for mo, mi, s, w in _PAIRS}

_GRP = []
for _m in range(_L + 1):
    if _m == 0:
        _GRP.append([l * l + l for l in range(_L + 1)])
    else:
        g = []
        for l in range(_m, _L + 1):
            g.append(l * l + l - _m)
            g.append(l * l + l + _m)
        _GRP.append(g)
_GK = [len(g) * _C for g in _GRP]


_BY_OUT = {}
for _mo, _mi, _s, _w in _PAIRS:
    _BY_OUT.setdefault(_mo, []).append((_mi, _s, _w))


def _wcast_body(w_ref, o_ref):
    o_ref[...] = w_ref[0].astype(jnp.bfloat16)


def _so2_body(x_ref, w_ref, o_ref):
    dn = (((1,), (0,)), ((), ()))
    for mo in range(_NO):
        acc = None
        for mi, s, wi in _BY_OUT[mo]:
            d = jax.lax.dot_general(x_ref[mi], w_ref[wi], dn,
                                    preferred_element_type=jnp.float32)
            if acc is None:
                acc = d if s > 0 else -d
            else:
                acc = acc + d if s > 0 else acc - d
        o_ref[mo] = acc


def kernel(x, weight):
    n = x.shape[0]
    tn = 128
    xt = jnp.transpose(x, (1, 0, 2)).astype(jnp.bfloat16)
    wb = pl.pallas_call(
        _wcast_body,
        in_specs=[pl.BlockSpec((1, _NW, _C, _C), lambda: (0, 0, 0, 0))],
        out_specs=pl.BlockSpec((_NW, _C, _C), lambda: (0, 0, 0)),
        out_shape=jax.ShapeDtypeStruct((_NW, _C, _C), jnp.bfloat16),
    )(weight)
    out = pl.pallas_call(
        _so2_body,
        grid=(n // tn,),
        in_specs=[
            pl.BlockSpec((_NO, tn, _C), lambda i: (0, i, 0)),
            pl.BlockSpec((_NW, _C, _C), lambda i: (0, 0, 0)),
        ],
        out_specs=pl.BlockSpec((_NO, tn, _C), lambda i: (0, i, 0)),
        out_shape=jax.ShapeDtypeStruct((_NO, n, _C), jnp.float32),
        compiler_params=pltpu.CompilerParams(
            dimension_semantics=("parallel",)),
    )(xt, wb)
    return jnp.transpose(out, (1, 0, 2))


# speedup vs baseline: 2.9029x; 1.1339x over previous
import numpy as np
import jax
import jax.numpy as jnp
from jax.experimental import pallas as pl
from jax.experimental.pallas import tpu as pltpu

_L = 6
_C = 128
_NO = (_L + 1) ** 2


def _so2_pair_table():
    ret = []
    widx = 0
    for lo in range(_L + 1):
        for li in range(_L + 1):
            mmax = min(lo, li)
            for mw in range(-mmax, mmax + 1):
                if mw != 0:
                    prs = ((-abs(mw), -mw), (abs(mw), mw))
                else:
                    prs = ((0, 0),)
                for mo, mi in prs:
                    ret.append((lo * lo + mo + lo, li * li + mi + li,
                                -1.0 if (mo > 0 and mi < 0) else 1.0, widx))
                widx += 1
    ret.sort()
    return ret, widx


_PAIRS, _NW = _so2_pair_table()
_PAIR_LUT = {(mi, mo): (s, w) for mo, mi, s, w in _PAIRS}

_GRP = []
for _m in range(_L + 1):
    if _m == 0:
        _GRP.append([l * l + l for l in range(_L + 1)])
    else:
        g = []
        for l in range(_m, _L + 1):
            g.append(l * l + l - _m)
            g.append(l * l + l + _m)
        _GRP.append(g)
_GK = [len(g) * _C for g in _GRP]


_BY_OUT = {}
for _mo, _mi, _s, _w in _PAIRS:
    _BY_OUT.setdefault(_mo, []).append((_mi, _s, _w))


def _so2_body(x_ref, w_ref, o_ref):
    dn = (((1,), (0,)), ((), ()))
    for mo in range(_NO):
        acc = None
        for mi, s, wi in _BY_OUT[mo]:
            d = jax.lax.dot_general(x_ref[mi], w_ref[0, wi], dn,
                                    preferred_element_type=jnp.float32)
            if acc is None:
                acc = d if s > 0 else -d
            else:
                acc = acc + d if s > 0 else acc - d
        o_ref[mo] = acc


def kernel(x, weight):
    n = x.shape[0]
    tn = 128
    xt = jnp.transpose(x, (1, 0, 2)).astype(jnp.bfloat16)
    out = pl.pallas_call(
        _so2_body,
        grid=(n // tn,),
        in_specs=[
            pl.BlockSpec((_NO, tn, _C), lambda i: (0, i, 0)),
            pl.BlockSpec((1, _NW, _C, _C), lambda i: (0, 0, 0, 0)),
        ],
        out_specs=pl.BlockSpec((_NO, tn, _C), lambda i: (0, i, 0)),
        out_shape=jax.ShapeDtypeStruct((_NO, n, _C), jnp.float32),
        compiler_params=pltpu.CompilerParams(
            dimension_semantics=("parallel",)),
    )(xt, weight)
    return jnp.transpose(out, (1, 0, 2))
